# Initial kernel scaffold; baseline (speedup 1.0000x reference)
import jax
import jax.numpy as jnp
from jax.experimental import pallas as pl


def kernel(x, pos_edge_index, neg_edge_index, W1, b1, W2, b2):
    raise NotImplementedError("write your pallas kernel here")


# speedup vs baseline: 8.0875x; 8.0875x over previous
import functools

import jax
import jax.numpy as jnp
from jax import lax
from jax.experimental import pallas as pl
from jax.experimental.pallas import tpu as pltpu
from jax.experimental.pallas import tpu_sc as plsc

N = 10000
D = 128
E = 320000
CHUNK = 128
NW = 32
RPW = 80
ROWS_PAD = NW * RPW
EPAD = ROWS_PAD * CHUNK
NP = 10240
NPT = NP // 16
NT = N // 16

_sc_mesh = plsc.VectorSubcoreMesh(core_axis_name="c", subcore_axis_name="s")
_sc_params = pltpu.CompilerParams(needs_layout_passes=False)


@functools.partial(
    pl.kernel,
    out_type=jax.ShapeDtypeStruct((2, NP), jnp.float32),
    mesh=_sc_mesh,
    compiler_params=_sc_params,
    scratch_types=[
        pltpu.VMEM((RPW, CHUNK), jnp.int32),
        pltpu.VMEM((NP,), jnp.float32),
        pltpu.VMEM((16, NPT), jnp.float32),
        pltpu.VMEM((NPT,), jnp.float32),
        pltpu.VMEM_SHARED((16, NP), jnp.float32),
    ],
)
def _deg_kernel(dst_hbm, degout_hbm, idx_v, hist_v, red_v, acc_v, shared):
    c = lax.axis_index("c")
    s = lax.axis_index("s")
    w = c * 16 + s
    zeros16 = jnp.zeros((16,), jnp.float32)
    ones16 = jnp.ones((16,), jnp.float32)

    def zbody(i, _):
        hist_v[pl.ds(i * 16, 16)] = zeros16
        return 0

    lax.fori_loop(0, NP // 16, zbody, 0)
    pltpu.sync_copy(dst_hbm.at[pl.ds(w * RPW, RPW)], idx_v)

    def ebody(j, _):
        for k in range(CHUNK // 16):
            idx = idx_v[j, pl.ds(k * 16, 16)]
            plsc.addupdate_scatter(hist_v, [idx], ones16)
        return 0

    lax.fori_loop(0, RPW, ebody, 0)
    pltpu.sync_copy(hist_v, shared.at[s])
    plsc.subcore_barrier()
    pltpu.sync_copy(shared.at[:, pl.ds(s * NPT, NPT)], red_v)

    def rbody(k, _):
        v = red_v[0, pl.ds(k * 16, 16)]
        for t in range(1, 16):
            v = v + red_v[t, pl.ds(k * 16, 16)]
        acc_v[pl.ds(k * 16, 16)] = v
        return 0

    lax.fori_loop(0, NPT // 16, rbody, 0)
    pltpu.sync_copy(acc_v, degout_hbm.at[c, pl.ds(s * NPT, NPT)])


@functools.partial(
    pl.kernel,
    out_type=jax.ShapeDtypeStruct((2, NP, D), jnp.float32),
    mesh=_sc_mesh,
    compiler_params=_sc_params,
    scratch_types=[
        pltpu.VMEM((RPW, CHUNK), jnp.int32),
        pltpu.VMEM((RPW, CHUNK), jnp.int32),
        pltpu.VMEM((CHUNK, D), jnp.float32),
        pltpu.VMEM_SHARED((NP, D), jnp.float32),
        pltpu.SemaphoreType.DMA,
    ],
)
def _spmm_kernel(y_hbm, src_hbm, dst_hbm, out_hbm, src_v, dst_v, buf, acc_sh, sem):
    c = lax.axis_index("c")
    s = lax.axis_index("s")
    w = c * 16 + s
    zeros16 = jnp.zeros((16,), jnp.float32)

    def zb(i, _):
        for k in range(D // 16):
            buf[i, pl.ds(k * 16, 16)] = zeros16
        return 0

    lax.fori_loop(0, CHUNK, zb, 0)
    for t in range(NPT // CHUNK):
        pltpu.sync_copy(buf, acc_sh.at[pl.ds(s * NPT + t * CHUNK, CHUNK)])
    pltpu.sync_copy(src_hbm.at[pl.ds(w * RPW, RPW)], src_v)
    pltpu.sync_copy(dst_hbm.at[pl.ds(w * RPW, RPW)], dst_v)
    plsc.subcore_barrier()

    def lbody(j, _):
        pltpu.async_copy(y_hbm.at[src_v.at[j]], buf, sem).wait()
        pltpu.sync_copy(buf, acc_sh.at[dst_v.at[j]], add=True)
        return 0

    lax.fori_loop(0, RPW, lbody, 0)
    plsc.subcore_barrier()
    pltpu.sync_copy(acc_sh.at[pl.ds(s * NPT, NPT)], out_hbm.at[c, pl.ds(s * NPT, NPT)])


def _mm1_body(x_ref, w_ref, degt_ref, o_ref):
    xw = jnp.dot(x_ref[...], w_ref[...], preferred_element_type=jnp.float32)
    deg = degt_ref[0:N, 0:1] + degt_ref[0:N, 1:2] + 1.0
    o_ref[...] = xw * lax.rsqrt(deg)


_mm1 = pl.pallas_call(
    _mm1_body, out_shape=jax.ShapeDtypeStruct((N, D), jnp.float32))


def _c2_body(part_ref, y1_ref, degt_ref, w_ref, b_ref, o_ref):
    deg = degt_ref[0:N, 0:1] + degt_ref[0:N, 1:2] + 1.0
    dinv = lax.rsqrt(deg)
    z = (part_ref[0, :N, :] + part_ref[1, :N, :] + y1_ref[...]) * dinv + b_ref[...]
    o_ref[...] = jnp.dot(z, w_ref[...], preferred_element_type=jnp.float32) * dinv


_c2 = pl.pallas_call(
    _c2_body, out_shape=jax.ShapeDtypeStruct((N, D), jnp.float32))


def _c3_body(part_ref, y2_ref, degt_ref, b_ref, o_ref):
    deg = degt_ref[0:N, 0:1] + degt_ref[0:N, 1:2] + 1.0
    dinv = lax.rsqrt(deg)
    o_ref[...] = (part_ref[0, :N, :] + part_ref[1, :N, :] + y2_ref[...]) * dinv + b_ref[...]


_c3 = pl.pallas_call(
    _c3_body, out_shape=jax.ShapeDtypeStruct((N, D), jnp.float32))


def kernel(x, pos_edge_index, neg_edge_index, W1, b1, W2, b2):
    ei = jnp.concatenate([pos_edge_index, neg_edge_index], axis=1).astype(jnp.int32)
    src, dst = ei[0], ei[1]
    pad = EPAD - E
    srcp = jnp.concatenate([src, jnp.zeros((pad,), jnp.int32)]).reshape(ROWS_PAD, CHUNK)
    dstp = jnp.concatenate([dst, jnp.full((pad,), N, jnp.int32)]).reshape(ROWS_PAD, CHUNK)

    degp = _deg_kernel(dstp)
    degt = degp.T
    b1r = b1.reshape(1, D)
    b2r = b2.reshape(1, D)

    y1 = _mm1(x, W1, degt)
    part1 = _spmm_kernel(y1, srcp, dstp)
    y2 = _c2(part1, y1, degt, W2, b1r)
    part2 = _spmm_kernel(y2, srcp, dstp)
    out = _c3(part2, y2, degt, b2r)
    return out


# speedup vs baseline: 10.2604x; 1.2687x over previous
import functools

import jax
import jax.numpy as jnp
from jax import lax
from jax.experimental import pallas as pl
from jax.experimental.pallas import tpu as pltpu
from jax.experimental.pallas import tpu_sc as plsc

N = 10000
D = 128
E = 320000
CHUNK = 128
NW = 32
RPW = 80
ROWS_PAD = NW * RPW
EPAD = ROWS_PAD * CHUNK
NP = 10112
NPT = NP // 16
NT = N // 16
NBUF = 2
HRPW = RPW // 2

_sc_mesh = plsc.VectorSubcoreMesh(core_axis_name="c", subcore_axis_name="s")
_sc_params = pltpu.CompilerParams(needs_layout_passes=False)


@functools.partial(
    pl.kernel,
    out_type=jax.ShapeDtypeStruct((NW, NP), jnp.float32),
    mesh=_sc_mesh,
    compiler_params=_sc_params,
    scratch_types=[
        pltpu.VMEM((RPW, CHUNK), jnp.int32),
        pltpu.VMEM((NP,), jnp.float32),
    ],
)
def _deg_kernel(dst_hbm, degout_hbm, idx_v, hist_v):
    c = lax.axis_index("c")
    s = lax.axis_index("s")
    w = c * 16 + s
    zeros16 = jnp.zeros((16,), jnp.float32)
    ones16 = jnp.ones((16,), jnp.float32)

    def zbody(i, _):
        hist_v[pl.ds(i * 16, 16)] = zeros16
        return 0

    lax.fori_loop(0, NP // 16, zbody, 0)
    pltpu.sync_copy(dst_hbm.at[pl.ds(w * RPW, RPW)], idx_v)

    def ebody(j, _):
        for k in range(CHUNK // 16):
            idx = idx_v[j, pl.ds(k * 16, 16)]
            plsc.addupdate_scatter(hist_v, [idx], ones16)
        return 0

    lax.fori_loop(0, RPW, ebody, 0)
    pltpu.sync_copy(hist_v, degout_hbm.at[w])


@functools.partial(
    pl.kernel,
    out_type=jax.ShapeDtypeStruct((2, NP, D), jnp.float32),
    mesh=_sc_mesh,
    compiler_params=_sc_params,
    scratch_types=[
        pltpu.VMEM((HRPW, CHUNK), jnp.int32),
        pltpu.VMEM((HRPW, CHUNK), jnp.int32),
        [pltpu.VMEM((CHUNK, D), jnp.float32) for _ in range(NBUF)],
        pltpu.VMEM_SHARED((NP, D), jnp.float32),
        [pltpu.SemaphoreType.DMA for _ in range(NBUF)],
    ],
)
def _spmm_kernel(y_hbm, src_hbm, dst_hbm, out_hbm, src_v, dst_v, bufs, acc_sh, gsems):
    c = lax.axis_index("c")
    s = lax.axis_index("s")
    w = c * 16 + s
    zeros16 = jnp.zeros((16,), jnp.float32)
    buf0 = bufs[0]

    def zb(i, _):
        for k in range(D // 16):
            buf0[i, pl.ds(k * 16, 16)] = zeros16
        return 0

    lax.fori_loop(0, CHUNK, zb, 0)
    for t in range(NPT // CHUNK):
        pltpu.sync_copy(buf0, acc_sh.at[pl.ds(s * NPT + t * CHUNK, CHUNK)])
    rem = NPT % CHUNK
    if rem:
        pltpu.sync_copy(buf0.at[pl.ds(0, rem)],
                        acc_sh.at[pl.ds(s * NPT + (NPT // CHUNK) * CHUNK, rem)])
    plsc.subcore_barrier()

    for h in range(RPW // HRPW):
        base = w * RPW + h * HRPW
        pltpu.sync_copy(src_hbm.at[pl.ds(base, HRPW)], src_v)
        pltpu.sync_copy(dst_hbm.at[pl.ds(base, HRPW)], dst_v)
        for b in range(NBUF):
            pltpu.async_copy(y_hbm.at[src_v.at[b]], bufs[b], gsems[b])

        def gbody(g, _):
            for b in range(NBUF):
                j = g * NBUF + b
                pltpu.make_async_copy(
                    y_hbm.at[pl.ds(0, CHUNK)], bufs[b], gsems[b]).wait()
                pltpu.sync_copy(bufs[b], acc_sh.at[dst_v.at[j]], add=True)
                jn = jnp.minimum(j + NBUF, HRPW - 1)
                pltpu.async_copy(y_hbm.at[src_v.at[jn]], bufs[b], gsems[b])
            return 0

        lax.fori_loop(0, HRPW // NBUF, gbody, 0)
        for b in range(NBUF):
            pltpu.make_async_copy(
                y_hbm.at[pl.ds(0, CHUNK)], bufs[b], gsems[b]).wait()
    plsc.subcore_barrier()
    pltpu.sync_copy(acc_sh.at[pl.ds(s * NPT, NPT)], out_hbm.at[c, pl.ds(s * NPT, NPT)])


def _mm1_body(x_ref, w_ref, degt_ref, o_ref):
    xw = jnp.dot(x_ref[...], w_ref[...], preferred_element_type=jnp.float32)
    deg = jnp.sum(degt_ref[0:N, :], axis=1, keepdims=True) + 1.0
    o_ref[...] = xw * lax.rsqrt(deg)


_mm1 = pl.pallas_call(
    _mm1_body, out_shape=jax.ShapeDtypeStruct((N, D), jnp.float32))


def _c2_body(part_ref, y1_ref, degt_ref, w_ref, b_ref, o_ref):
    deg = jnp.sum(degt_ref[0:N, :], axis=1, keepdims=True) + 1.0
    dinv = lax.rsqrt(deg)
    z = (part_ref[0, :N, :] + part_ref[1, :N, :] + y1_ref[...]) * dinv + b_ref[...]
    o_ref[...] = jnp.dot(z, w_ref[...], preferred_element_type=jnp.float32) * dinv


_c2 = pl.pallas_call(
    _c2_body, out_shape=jax.ShapeDtypeStruct((N, D), jnp.float32))


def _c3_body(part_ref, y2_ref, degt_ref, b_ref, o_ref):
    deg = jnp.sum(degt_ref[0:N, :], axis=1, keepdims=True) + 1.0
    dinv = lax.rsqrt(deg)
    o_ref[...] = (part_ref[0, :N, :] + part_ref[1, :N, :] + y2_ref[...]) * dinv + b_ref[...]


_c3 = pl.pallas_call(
    _c3_body, out_shape=jax.ShapeDtypeStruct((N, D), jnp.float32))


def kernel(x, pos_edge_index, neg_edge_index, W1, b1, W2, b2):
    ei = jnp.concatenate([pos_edge_index, neg_edge_index], axis=1).astype(jnp.int32)
    src, dst = ei[0], ei[1]
    pad = EPAD - E
    srcp = jnp.concatenate([src, jnp.zeros((pad,), jnp.int32)]).reshape(ROWS_PAD, CHUNK)
    dstp = jnp.concatenate([dst, jnp.full((pad,), N, jnp.int32)]).reshape(ROWS_PAD, CHUNK)

    degp = _deg_kernel(dstp)
    degt = degp.T
    b1r = b1.reshape(1, D)
    b2r = b2.reshape(1, D)

    y1 = _mm1(x, W1, degt)
    part1 = _spmm_kernel(y1, srcp, dstp)
    y2 = _c2(part1, y1, degt, W2, b1r)
    part2 = _spmm_kernel(y2, srcp, dstp)
    out = _c3(part2, y2, degt, b2r)
    return out


# speedup vs baseline: 10.4000x; 1.0136x over previous
import functools

import jax
import jax.numpy as jnp
from jax import lax
from jax.experimental import pallas as pl
from jax.experimental.pallas import tpu as pltpu
from jax.experimental.pallas import tpu_sc as plsc

N = 10000
D = 128
E = 320000
CHUNK = 128
NW = 32
RPW = 80
ROWS_PAD = NW * RPW
EPAD = ROWS_PAD * CHUNK
NP = 10112
NPT = NP // 16
NT = N // 16
NBUF = 2
HRPW = RPW // 2
SPLIT = 2
SC_ = CHUNK // SPLIT

_sc_mesh = plsc.VectorSubcoreMesh(core_axis_name="c", subcore_axis_name="s")
_sc_params = pltpu.CompilerParams(needs_layout_passes=False)


@functools.partial(
    pl.kernel,
    out_type=jax.ShapeDtypeStruct((NW, NP), jnp.float32),
    mesh=_sc_mesh,
    compiler_params=_sc_params,
    scratch_types=[
        pltpu.VMEM((RPW, CHUNK), jnp.int32),
        pltpu.VMEM((NP,), jnp.float32),
    ],
)
def _deg_kernel(dst_hbm, degout_hbm, idx_v, hist_v):
    c = lax.axis_index("c")
    s = lax.axis_index("s")
    w = c * 16 + s
    zeros16 = jnp.zeros((16,), jnp.float32)
    ones16 = jnp.ones((16,), jnp.float32)

    def zbody(i, _):
        hist_v[pl.ds(i * 16, 16)] = zeros16
        return 0

    lax.fori_loop(0, NP // 16, zbody, 0)
    pltpu.sync_copy(dst_hbm.at[pl.ds(w * RPW, RPW)], idx_v)

    def ebody(j, _):
        for k in range(CHUNK // 16):
            idx = idx_v[j, pl.ds(k * 16, 16)]
            plsc.addupdate_scatter(hist_v, [idx], ones16)
        return 0

    lax.fori_loop(0, RPW, ebody, 0)
    pltpu.sync_copy(hist_v, degout_hbm.at[w])


@functools.partial(
    pl.kernel,
    out_type=jax.ShapeDtypeStruct((2, NP, D), jnp.float32),
    mesh=_sc_mesh,
    compiler_params=_sc_params,
    scratch_types=[
        pltpu.VMEM((HRPW, CHUNK), jnp.int32),
        pltpu.VMEM((HRPW, CHUNK), jnp.int32),
        [pltpu.VMEM((CHUNK, D), jnp.float32) for _ in range(NBUF)],
        pltpu.VMEM_SHARED((NP, D), jnp.float32),
        [pltpu.SemaphoreType.DMA for _ in range(NBUF)],
    ],
)
def _spmm_kernel(y_hbm, src_hbm, dst_hbm, out_hbm, src_v, dst_v, bufs, acc_sh, gsems):
    c = lax.axis_index("c")
    s = lax.axis_index("s")
    w = c * 16 + s
    zeros16 = jnp.zeros((16,), jnp.float32)
    buf0 = bufs[0]

    def zb(i, _):
        for k in range(D // 16):
            buf0[i, pl.ds(k * 16, 16)] = zeros16
        return 0

    lax.fori_loop(0, CHUNK, zb, 0)
    for t in range(NPT // CHUNK):
        pltpu.sync_copy(buf0, acc_sh.at[pl.ds(s * NPT + t * CHUNK, CHUNK)])
    rem = NPT % CHUNK
    if rem:
        pltpu.sync_copy(buf0.at[pl.ds(0, rem)],
                        acc_sh.at[pl.ds(s * NPT + (NPT // CHUNK) * CHUNK, rem)])
    plsc.subcore_barrier()

    for h in range(RPW // HRPW):
        base = w * RPW + h * HRPW
        pltpu.sync_copy(src_hbm.at[pl.ds(base, HRPW)], src_v)
        pltpu.sync_copy(dst_hbm.at[pl.ds(base, HRPW)], dst_v)
        def fire_gather(j, b):
            for p in range(SPLIT):
                pltpu.async_copy(
                    y_hbm.at[src_v.at[j, pl.ds(p * SC_, SC_)]],
                    bufs[b].at[pl.ds(p * SC_, SC_)], gsems[b])

        def drain_gather(b):
            for p in range(SPLIT):
                pltpu.make_async_copy(
                    y_hbm.at[pl.ds(0, SC_)],
                    bufs[b].at[pl.ds(p * SC_, SC_)], gsems[b]).wait()

        for b in range(NBUF):
            fire_gather(b, b)

        def gbody(g, _):
            for b in range(NBUF):
                j = g * NBUF + b
                drain_gather(b)
                pltpu.sync_copy(bufs[b], acc_sh.at[dst_v.at[j]], add=True)
                jn = jnp.minimum(j + NBUF, HRPW - 1)
                fire_gather(jn, b)
            return 0

        lax.fori_loop(0, HRPW // NBUF, gbody, 0)
        for b in range(NBUF):
            drain_gather(b)
    plsc.subcore_barrier()
    pltpu.sync_copy(acc_sh.at[pl.ds(s * NPT, NPT)], out_hbm.at[c, pl.ds(s * NPT, NPT)])


def _mm1_body(x_ref, w_ref, degt_ref, o_ref):
    xw = jnp.dot(x_ref[...], w_ref[...], preferred_element_type=jnp.float32)
    deg = jnp.sum(degt_ref[0:N, :], axis=1, keepdims=True) + 1.0
    o_ref[...] = xw * lax.rsqrt(deg)


_mm1 = pl.pallas_call(
    _mm1_body, out_shape=jax.ShapeDtypeStruct((N, D), jnp.float32))


def _c2_body(part_ref, y1_ref, degt_ref, w_ref, b_ref, o_ref):
    deg = jnp.sum(degt_ref[0:N, :], axis=1, keepdims=True) + 1.0
    dinv = lax.rsqrt(deg)
    z = (part_ref[0, :N, :] + part_ref[1, :N, :] + y1_ref[...]) * dinv + b_ref[...]
    o_ref[...] = jnp.dot(z, w_ref[...], preferred_element_type=jnp.float32) * dinv


_c2 = pl.pallas_call(
    _c2_body, out_shape=jax.ShapeDtypeStruct((N, D), jnp.float32))


def _c3_body(part_ref, y2_ref, degt_ref, b_ref, o_ref):
    deg = jnp.sum(degt_ref[0:N, :], axis=1, keepdims=True) + 1.0
    dinv = lax.rsqrt(deg)
    o_ref[...] = (part_ref[0, :N, :] + part_ref[1, :N, :] + y2_ref[...]) * dinv + b_ref[...]


_c3 = pl.pallas_call(
    _c3_body, out_shape=jax.ShapeDtypeStruct((N, D), jnp.float32))


def kernel(x, pos_edge_index, neg_edge_index, W1, b1, W2, b2):
    ei = jnp.concatenate([pos_edge_index, neg_edge_index], axis=1).astype(jnp.int32)
    src, dst = ei[0], ei[1]
    pad = EPAD - E
    srcp = jnp.concatenate([src, jnp.zeros((pad,), jnp.int32)]).reshape(ROWS_PAD, CHUNK)
    dstp = jnp.concatenate([dst, jnp.full((pad,), N, jnp.int32)]).reshape(ROWS_PAD, CHUNK)

    degp = _deg_kernel(dstp)
    degt = degp.T
    b1r = b1.reshape(1, D)
    b2r = b2.reshape(1, D)

    y1 = _mm1(x, W1, degt)
    part1 = _spmm_kernel(y1, srcp, dstp)
    y2 = _c2(part1, y1, degt, W2, b1r)
    part2 = _spmm_kernel(y2, srcp, dstp)
    out = _c3(part2, y2, degt, b2r)
    return out


# speedup vs baseline: 14.7778x; 1.4209x over previous
import functools

import jax
import jax.numpy as jnp
from jax import lax
from jax.experimental import pallas as pl
from jax.experimental.pallas import tpu as pltpu
from jax.experimental.pallas import tpu_sc as plsc

N = 10000
D = 128
E = 320000
CHUNK = 128
NW = 32
RPW = 80
ROWS_PAD = NW * RPW
EPAD = ROWS_PAD * CHUNK
NP = 10112
NPT = NP // 16
NT = N // 16
HALF = 5056
ACC = 5064
C32 = 32
BLK = 8
ROWS32 = EPAD // C32
CPT = ROWS32 // 16
NSB = CPT // (2 * BLK)

_sc_mesh = plsc.VectorSubcoreMesh(core_axis_name="c", subcore_axis_name="s")
_sc_params = pltpu.CompilerParams(needs_layout_passes=False)


@functools.partial(
    pl.kernel,
    out_type=jax.ShapeDtypeStruct((NW, NP), jnp.float32),
    mesh=_sc_mesh,
    compiler_params=_sc_params,
    scratch_types=[
        pltpu.VMEM((RPW, CHUNK), jnp.int32),
        pltpu.VMEM((NP,), jnp.float32),
    ],
)
def _deg_kernel(dst_hbm, degout_hbm, idx_v, hist_v):
    c = lax.axis_index("c")
    s = lax.axis_index("s")
    w = c * 16 + s
    zeros16 = jnp.zeros((16,), jnp.float32)
    ones16 = jnp.ones((16,), jnp.float32)

    def zbody(i, _):
        hist_v[pl.ds(i * 16, 16)] = zeros16
        return 0

    lax.fori_loop(0, NP // 16, zbody, 0)
    pltpu.sync_copy(dst_hbm.at[pl.ds(w * RPW, RPW)], idx_v)

    def ebody(j, _):
        for k in range(CHUNK // 16):
            idx = idx_v[j, pl.ds(k * 16, 16)]
            plsc.addupdate_scatter(hist_v, [idx], ones16)
        return 0

    lax.fori_loop(0, RPW, ebody, 0)
    pltpu.sync_copy(hist_v, degout_hbm.at[w])


@functools.partial(
    pl.kernel,
    out_type=jax.ShapeDtypeStruct((2, ACC, D), jnp.float32),
    mesh=_sc_mesh,
    compiler_params=_sc_params,
    scratch_types=[
        pltpu.VMEM_SHARED((N, D), jnp.float32),
        pltpu.VMEM_SHARED((ACC, D), jnp.float32),
        [pltpu.VMEM((C32, D), jnp.float32) for _ in range(2)],
        [pltpu.VMEM((BLK, 2 * C32), jnp.int32) for _ in range(2)],
        pltpu.VMEM((1, C32), jnp.int32),
        [pltpu.SemaphoreType.DMA for _ in range(2)],
        pltpu.SemaphoreType.DMA,
    ],
)
def _spmm_kernel(y_hbm, sd_hbm, out_hbm, y_sh, acc_sh, bufs, sdblks, dl,
                 gsems, isem):
    c = lax.axis_index("c")
    s = lax.axis_index("s")
    zeros16 = jnp.zeros((16,), jnp.float32)
    buf0 = bufs[0]

    def zb(i, _):
        for k in range(D // 16):
            buf0[i, pl.ds(k * 16, 16)] = zeros16
        return 0

    lax.fori_loop(0, C32, zb, 0)
    for t in range(9):
        pltpu.sync_copy(buf0, acc_sh.at[pl.ds(s * 312 + t * C32, C32)])
    pltpu.sync_copy(buf0.at[pl.ds(0, 24)], acc_sh.at[pl.ds(s * 312 + 288, 24)])

    @pl.when(s < 9)
    def _zero_tail():
        pltpu.sync_copy(buf0.at[pl.ds(0, 8)], acc_sh.at[pl.ds(4992 + s * 8, 8)])

    pltpu.sync_copy(y_hbm.at[pl.ds(s * 624, 624)], y_sh.at[pl.ds(s * 624, 624)])

    @pl.when(s == 0)
    def _stage_tail():
        pltpu.sync_copy(y_hbm.at[pl.ds(9984, 16)], y_sh.at[pl.ds(9984, 16)])

    plsc.subcore_barrier()

    base = s * CPT // BLK * BLK
    coff = c * HALF

    pltpu.sync_copy(sd_hbm.at[pl.ds(base, BLK)], sdblks[0])
    pltpu.sync_copy(sd_hbm.at[pl.ds(base + BLK, BLK)], sdblks[1])
    pltpu.async_copy(sd_hbm.at[pl.ds(base + BLK, BLK)], sdblks[1], isem)
    for b in range(2):
        pltpu.async_copy(y_sh.at[sdblks[0].at[b, pl.ds(0, C32)]], bufs[b],
                         gsems[b])

    def sb_body(m, _):
        for k in range(16):
            b = k % 2
            slot = k // BLK
            r = k % BLK
            if k == 5:
                pltpu.make_async_copy(
                    sd_hbm.at[pl.ds(0, BLK)], sdblks[1], isem).wait()
            if k == 8:
                gn = jnp.minimum(2 * m + 2, 2 * NSB - 1)
                pltpu.async_copy(
                    sd_hbm.at[pl.ds(base + gn * BLK, BLK)], sdblks[0], isem)
            if k == 13:
                pltpu.make_async_copy(
                    sd_hbm.at[pl.ds(0, BLK)], sdblks[0], isem).wait()
            pltpu.make_async_copy(
                y_hbm.at[pl.ds(0, C32)], bufs[b], gsems[b]).wait()
            for q in range(2):
                d = sdblks[slot][r, pl.ds(C32 + q * 16, 16)]
                loc = d - coff
                bad = (loc < 0) | (loc >= HALF)
                dl[0, pl.ds(q * 16, 16)] = jnp.where(bad, HALF, loc)
            pltpu.sync_copy(bufs[b], acc_sh.at[dl.at[0]], add=True)
            if k < 6:
                nref, nrow = sdblks[0], k + 2
            elif k < 14:
                nref, nrow = sdblks[1], k - 6
            else:
                nref, nrow = sdblks[0], k - 14
            pltpu.async_copy(y_sh.at[nref.at[nrow, pl.ds(0, C32)]], bufs[b],
                             gsems[b])
        gn = jnp.minimum(2 * m + 3, 2 * NSB - 1)
        pltpu.async_copy(sd_hbm.at[pl.ds(base + gn * BLK, BLK)], sdblks[1], isem)
        return 0

    lax.fori_loop(0, NSB, sb_body, 0)
    pltpu.make_async_copy(sd_hbm.at[pl.ds(0, BLK)], sdblks[1], isem).wait()
    for b in range(2):
        pltpu.make_async_copy(y_hbm.at[pl.ds(0, C32)], bufs[b], gsems[b]).wait()
    plsc.subcore_barrier()
    pltpu.sync_copy(acc_sh.at[pl.ds(s * 312, 312)],
                    out_hbm.at[c, pl.ds(s * 312, 312)])

    @pl.when(s < 9)
    def _copy_tail():
        pltpu.sync_copy(acc_sh.at[pl.ds(4992 + s * 8, 8)],
                        out_hbm.at[c, pl.ds(4992 + s * 8, 8)])


def _mm1_body(x_ref, w_ref, degt_ref, o_ref):
    xw = jnp.dot(x_ref[...], w_ref[...], preferred_element_type=jnp.float32)
    deg = jnp.sum(degt_ref[0:N, :], axis=1, keepdims=True) + 1.0
    o_ref[...] = xw * lax.rsqrt(deg)


_mm1 = pl.pallas_call(
    _mm1_body, out_shape=jax.ShapeDtypeStruct((N, D), jnp.float32))


def _c2_body(part_ref, y1_ref, degt_ref, w_ref, b_ref, o_ref):
    deg = jnp.sum(degt_ref[0:N, :], axis=1, keepdims=True) + 1.0
    dinv = lax.rsqrt(deg)
    agg = jnp.concatenate(
        [part_ref[0, 0:HALF, :], part_ref[1, 0:(N - HALF), :]], axis=0)
    z = (agg + y1_ref[...]) * dinv + b_ref[...]
    o_ref[...] = jnp.dot(z, w_ref[...], preferred_element_type=jnp.float32) * dinv


_c2 = pl.pallas_call(
    _c2_body, out_shape=jax.ShapeDtypeStruct((N, D), jnp.float32))


def _c3_body(part_ref, y2_ref, degt_ref, b_ref, o_ref):
    deg = jnp.sum(degt_ref[0:N, :], axis=1, keepdims=True) + 1.0
    dinv = lax.rsqrt(deg)
    agg = jnp.concatenate(
        [part_ref[0, 0:HALF, :], part_ref[1, 0:(N - HALF), :]], axis=0)
    o_ref[...] = (agg + y2_ref[...]) * dinv + b_ref[...]


_c3 = pl.pallas_call(
    _c3_body, out_shape=jax.ShapeDtypeStruct((N, D), jnp.float32))


def kernel(x, pos_edge_index, neg_edge_index, W1, b1, W2, b2):
    ei = jnp.concatenate([pos_edge_index, neg_edge_index], axis=1).astype(jnp.int32)
    src, dst = ei[0], ei[1]
    pad = EPAD - E
    srcaliflat = jnp.concatenate([src, jnp.zeros((pad,), jnp.int32)])
    dstflat = jnp.concatenate([dst, jnp.full((pad,), N, jnp.int32)])
    dstp = dstflat.reshape(ROWS_PAD, CHUNK)
    sd64 = jnp.concatenate([srcaliflat.reshape(ROWS32, C32),
                            dstflat.reshape(ROWS32, C32)], axis=1)

    degp = _deg_kernel(dstp)
    degt = degp.T
    b1r = b1.reshape(1, D)
    b2r = b2.reshape(1, D)

    y1 = _mm1(x, W1, degt)
    part1 = _spmm_kernel(y1, sd64)
    y2 = _c2(part1, y1, degt, W2, b1r)
    part2 = _spmm_kernel(y2, sd64)
    out = _c3(part2, y2, degt, b2r)
    return out


# speedup vs baseline: 15.0583x; 1.0190x over previous
import functools

import jax
import jax.numpy as jnp
from jax import lax
from jax.experimental import pallas as pl
from jax.experimental.pallas import tpu as pltpu
from jax.experimental.pallas import tpu_sc as plsc

N = 10000
D = 128
E = 320000
CHUNK = 128
NW = 32
RPW = 80
ROWS_PAD = NW * RPW
EPAD = ROWS_PAD * CHUNK
NP = 10112
NPT = NP // 16
NT = N // 16
HALF = 5056
ACC = 5064
C32 = 32
BLK = 8
ROWS32 = EPAD // C32
CPT = ROWS32 // 16
NSB = CPT // (2 * BLK)

_sc_mesh = plsc.VectorSubcoreMesh(core_axis_name="c", subcore_axis_name="s")
_sc_params = pltpu.CompilerParams(needs_layout_passes=False)


@functools.partial(
    pl.kernel,
    out_type=jax.ShapeDtypeStruct((NW, NP), jnp.float32),
    mesh=_sc_mesh,
    compiler_params=_sc_params,
    scratch_types=[
        pltpu.VMEM((RPW, CHUNK), jnp.int32),
        pltpu.VMEM((NP,), jnp.float32),
    ],
)
def _deg_kernel(dst_hbm, degout_hbm, idx_v, hist_v):
    c = lax.axis_index("c")
    s = lax.axis_index("s")
    w = c * 16 + s
    zeros16 = jnp.zeros((16,), jnp.float32)
    ones16 = jnp.ones((16,), jnp.float32)

    def zbody(i, _):
        hist_v[pl.ds(i * 16, 16)] = zeros16
        return 0

    lax.fori_loop(0, NP // 16, zbody, 0)
    pltpu.sync_copy(dst_hbm.at[pl.ds(w * RPW, RPW)], idx_v)

    def ebody(j, _):
        for k in range(CHUNK // 16):
            idx = idx_v[j, pl.ds(k * 16, 16)]
            plsc.addupdate_scatter(hist_v, [idx], ones16)
        return 0

    lax.fori_loop(0, RPW, ebody, 0)
    pltpu.sync_copy(hist_v, degout_hbm.at[w])


@functools.partial(
    pl.kernel,
    out_type=jax.ShapeDtypeStruct((2, ACC, D), jnp.float32),
    mesh=_sc_mesh,
    compiler_params=_sc_params,
    scratch_types=[
        pltpu.VMEM_SHARED((N, D), jnp.float32),
        pltpu.VMEM_SHARED((ACC, D), jnp.float32),
        [pltpu.VMEM((C32, D), jnp.float32) for _ in range(2)],
        [pltpu.VMEM((BLK, 2 * C32), jnp.int32) for _ in range(2)],
        [pltpu.VMEM((1, C32), jnp.int32) for _ in range(2)],
        [pltpu.SemaphoreType.DMA for _ in range(2)],
        [pltpu.SemaphoreType.DMA for _ in range(2)],
        pltpu.SemaphoreType.DMA,
    ],
)
def _spmm_kernel(y_hbm, sd_hbm, out_hbm, y_sh, acc_sh, bufs, sdblks, dls,
                 gsems, ssems, isem):
    c = lax.axis_index("c")
    s = lax.axis_index("s")
    zeros16 = jnp.zeros((16,), jnp.float32)
    buf0 = bufs[0]

    def zb(i, _):
        for k in range(D // 16):
            buf0[i, pl.ds(k * 16, 16)] = zeros16
        return 0

    lax.fori_loop(0, C32, zb, 0)
    for t in range(9):
        pltpu.sync_copy(buf0, acc_sh.at[pl.ds(s * 312 + t * C32, C32)])
    pltpu.sync_copy(buf0.at[pl.ds(0, 24)], acc_sh.at[pl.ds(s * 312 + 288, 24)])

    @pl.when(s < 9)
    def _zero_tail():
        pltpu.sync_copy(buf0.at[pl.ds(0, 8)], acc_sh.at[pl.ds(4992 + s * 8, 8)])

    pltpu.sync_copy(y_hbm.at[pl.ds(s * 624, 624)], y_sh.at[pl.ds(s * 624, 624)])

    @pl.when(s == 0)
    def _stage_tail():
        pltpu.sync_copy(y_hbm.at[pl.ds(9984, 16)], y_sh.at[pl.ds(9984, 16)])

    plsc.subcore_barrier()

    base = s * CPT // BLK * BLK
    coff = c * HALF

    pltpu.sync_copy(sd_hbm.at[pl.ds(base, BLK)], sdblks[0])
    pltpu.sync_copy(sd_hbm.at[pl.ds(base + BLK, BLK)], sdblks[1])
    pltpu.async_copy(sd_hbm.at[pl.ds(base + BLK, BLK)], sdblks[1], isem)
    pltpu.async_copy(y_sh.at[sdblks[0].at[0, pl.ds(0, C32)]], bufs[0], gsems[0])

    def sb_body(m, _):
        for k in range(16):
            b = k % 2
            nb = (k + 1) % 2
            slot = k // BLK
            r = k % BLK
            if k == 5:
                pltpu.make_async_copy(
                    sd_hbm.at[pl.ds(0, BLK)], sdblks[1], isem).wait()
            if k == 8:
                gn = jnp.minimum(2 * m + 2, 2 * NSB - 1)
                pltpu.async_copy(
                    sd_hbm.at[pl.ds(base + gn * BLK, BLK)], sdblks[0], isem)
            if k == 13:
                pltpu.make_async_copy(
                    sd_hbm.at[pl.ds(0, BLK)], sdblks[0], isem).wait()
            pltpu.make_async_copy(
                y_hbm.at[pl.ds(0, C32)], bufs[b], gsems[b]).wait()
            for q in range(2):
                d = sdblks[slot][r, pl.ds(C32 + q * 16, 16)]
                loc = d - coff
                bad = (loc < 0) | (loc >= HALF)
                dls[b][0, pl.ds(q * 16, 16)] = jnp.where(bad, HALF, loc)
            pltpu.async_copy(bufs[b], acc_sh.at[dls[b].at[0]], ssems[b],
                             add=True)
            if k == 0:
                @pl.when(m > 0)
                def _drain_first():
                    pltpu.make_async_copy(
                        y_hbm.at[pl.ds(0, C32)], bufs[nb], ssems[nb]).wait()
            else:
                pltpu.make_async_copy(
                    y_hbm.at[pl.ds(0, C32)], bufs[nb], ssems[nb]).wait()
            if k < 7:
                nref, nrow = sdblks[0], k + 1
            elif k < 15:
                nref, nrow = sdblks[1], k - 7
            else:
                nref, nrow = sdblks[0], 0
            pltpu.async_copy(y_sh.at[nref.at[nrow, pl.ds(0, C32)]], bufs[nb],
                             gsems[nb])
        gn = jnp.minimum(2 * m + 3, 2 * NSB - 1)
        pltpu.async_copy(sd_hbm.at[pl.ds(base + gn * BLK, BLK)], sdblks[1], isem)
        return 0

    lax.fori_loop(0, NSB, sb_body, 0)
    pltpu.make_async_copy(sd_hbm.at[pl.ds(0, BLK)], sdblks[1], isem).wait()
    pltpu.make_async_copy(y_hbm.at[pl.ds(0, C32)], bufs[1], ssems[1]).wait()
    pltpu.make_async_copy(y_hbm.at[pl.ds(0, C32)], bufs[0], gsems[0]).wait()
    plsc.subcore_barrier()
    pltpu.sync_copy(acc_sh.at[pl.ds(s * 312, 312)],
                    out_hbm.at[c, pl.ds(s * 312, 312)])

    @pl.when(s < 9)
    def _copy_tail():
        pltpu.sync_copy(acc_sh.at[pl.ds(4992 + s * 8, 8)],
                        out_hbm.at[c, pl.ds(4992 + s * 8, 8)])


def _mm1_body(x_ref, w_ref, degt_ref, o_ref):
    xw = jnp.dot(x_ref[...], w_ref[...], preferred_element_type=jnp.float32)
    deg = jnp.sum(degt_ref[0:N, :], axis=1, keepdims=True) + 1.0
    o_ref[...] = xw * lax.rsqrt(deg)


_mm1 = pl.pallas_call(
    _mm1_body, out_shape=jax.ShapeDtypeStruct((N, D), jnp.float32))


def _c2_body(part_ref, y1_ref, degt_ref, w_ref, b_ref, o_ref):
    deg = jnp.sum(degt_ref[0:N, :], axis=1, keepdims=True) + 1.0
    dinv = lax.rsqrt(deg)
    agg = jnp.concatenate(
        [part_ref[0, 0:HALF, :], part_ref[1, 0:(N - HALF), :]], axis=0)
    z = (agg + y1_ref[...]) * dinv + b_ref[...]
    o_ref[...] = jnp.dot(z, w_ref[...], preferred_element_type=jnp.float32) * dinv


_c2 = pl.pallas_call(
    _c2_body, out_shape=jax.ShapeDtypeStruct((N, D), jnp.float32))


def _c3_body(part_ref, y2_ref, degt_ref, b_ref, o_ref):
    deg = jnp.sum(degt_ref[0:N, :], axis=1, keepdims=True) + 1.0
    dinv = lax.rsqrt(deg)
    agg = jnp.concatenate(
        [part_ref[0, 0:HALF, :], part_ref[1, 0:(N - HALF), :]], axis=0)
    o_ref[...] = (agg + y2_ref[...]) * dinv + b_ref[...]


_c3 = pl.pallas_call(
    _c3_body, out_shape=jax.ShapeDtypeStruct((N, D), jnp.float32))


def kernel(x, pos_edge_index, neg_edge_index, W1, b1, W2, b2):
    ei = jnp.concatenate([pos_edge_index, neg_edge_index], axis=1).astype(jnp.int32)
    src, dst = ei[0], ei[1]
    pad = EPAD - E
    srcaliflat = jnp.concatenate([src, jnp.zeros((pad,), jnp.int32)])
    dstflat = jnp.concatenate([dst, jnp.full((pad,), N, jnp.int32)])
    dstp = dstflat.reshape(ROWS_PAD, CHUNK)
    sd64 = jnp.concatenate([srcaliflat.reshape(ROWS32, C32),
                            dstflat.reshape(ROWS32, C32)], axis=1)

    degp = _deg_kernel(dstp)
    degt = degp.T
    b1r = b1.reshape(1, D)
    b2r = b2.reshape(1, D)

    y1 = _mm1(x, W1, degt)
    part1 = _spmm_kernel(y1, sd64)
    y2 = _c2(part1, y1, degt, W2, b1r)
    part2 = _spmm_kernel(y2, sd64)
    out = _c3(part2, y2, degt, b2r)
    return out


# speedup vs baseline: 15.0588x; 1.0000x over previous
import functools

import jax
import jax.numpy as jnp
from jax import lax
from jax.experimental import pallas as pl
from jax.experimental.pallas import tpu as pltpu
from jax.experimental.pallas import tpu_sc as plsc

N = 10000
D = 128
E = 320000
CHUNK = 128
NW = 32
RPW = 80
ROWS_PAD = NW * RPW
EPAD = ROWS_PAD * CHUNK
NP = 10112
HALF = 5056
ACC = 5064
C32 = 32
BLK = 8
ROWS32 = EPAD // C32
CPT = ROWS32 // 16
NSB = CPT // (2 * BLK)

_sc_mesh = plsc.VectorSubcoreMesh(core_axis_name="c", subcore_axis_name="s")
_sc_params = pltpu.CompilerParams(needs_layout_passes=False)


@functools.partial(
    pl.kernel,
    out_type=jax.ShapeDtypeStruct((NW, NP), jnp.float32),
    mesh=_sc_mesh,
    compiler_params=_sc_params,
    scratch_types=[
        pltpu.VMEM((RPW, CHUNK), jnp.int32),
        pltpu.VMEM((NP,), jnp.float32),
    ],
)
def _deg_kernel(dst_hbm, degout_hbm, idx_v, hist_v):
    c = lax.axis_index("c")
    s = lax.axis_index("s")
    w = c * 16 + s
    zeros16 = jnp.zeros((16,), jnp.float32)
    ones16 = jnp.ones((16,), jnp.float32)

    def zbody(i, _):
        hist_v[pl.ds(i * 16, 16)] = zeros16
        return 0

    lax.fori_loop(0, NP // 16, zbody, 0)
    pltpu.sync_copy(dst_hbm.at[pl.ds(w * RPW, RPW)], idx_v)

    def ebody(j, _):
        for k in range(CHUNK // 16):
            idx = idx_v[j, pl.ds(k * 16, 16)]
            plsc.addupdate_scatter(hist_v, [idx], ones16)
        return 0

    lax.fori_loop(0, RPW, ebody, 0)
    pltpu.sync_copy(hist_v, degout_hbm.at[w])


@functools.partial(
    pl.kernel,
    out_type=jax.ShapeDtypeStruct((2, ACC, D), jnp.float32),
    mesh=_sc_mesh,
    compiler_params=_sc_params,
    scratch_types=[
        pltpu.VMEM_SHARED((N, D), jnp.float32),
        pltpu.VMEM_SHARED((ACC, D), jnp.float32),
        [pltpu.VMEM((C32, D), jnp.float32) for _ in range(2)],
        [pltpu.VMEM((BLK, 2 * C32), jnp.int32) for _ in range(2)],
        [pltpu.VMEM((1, C32), jnp.int32) for _ in range(2)],
        [pltpu.SemaphoreType.DMA for _ in range(2)],
        [pltpu.SemaphoreType.DMA for _ in range(2)],
        pltpu.SemaphoreType.DMA,
    ],
)
def _spmm_kernel(y_hbm, sd_hbm, out_hbm, y_sh, acc_sh, bufs, sdblks, dls,
                 gsems, ssems, isem):
    c = lax.axis_index("c")
    s = lax.axis_index("s")
    zeros16 = jnp.zeros((16,), jnp.float32)
    buf0 = bufs[0]

    def zb(i, _):
        for k in range(D // 16):
            buf0[i, pl.ds(k * 16, 16)] = zeros16
        return 0

    lax.fori_loop(0, C32, zb, 0)
    for t in range(9):
        pltpu.sync_copy(buf0, acc_sh.at[pl.ds(s * 312 + t * C32, C32)])
    pltpu.sync_copy(buf0.at[pl.ds(0, 24)], acc_sh.at[pl.ds(s * 312 + 288, 24)])

    @pl.when(s < 9)
    def _zero_tail():
        pltpu.sync_copy(buf0.at[pl.ds(0, 8)], acc_sh.at[pl.ds(4992 + s * 8, 8)])

    pltpu.sync_copy(y_hbm.at[pl.ds(s * 624, 624)], y_sh.at[pl.ds(s * 624, 624)])

    @pl.when(s == 0)
    def _stage_tail():
        pltpu.sync_copy(y_hbm.at[pl.ds(9984, 16)], y_sh.at[pl.ds(9984, 16)])

    plsc.subcore_barrier()

    base = s * CPT // BLK * BLK
    coff = c * HALF

    pltpu.sync_copy(sd_hbm.at[pl.ds(base, BLK)], sdblks[0])
    pltpu.sync_copy(sd_hbm.at[pl.ds(base + BLK, BLK)], sdblks[1])
    pltpu.async_copy(sd_hbm.at[pl.ds(base + BLK, BLK)], sdblks[1], isem)
    pltpu.async_copy(y_sh.at[sdblks[0].at[0, pl.ds(0, C32)]], bufs[0], gsems[0])

    def sb_body(m, _):
        for k in range(16):
            b = k % 2
            nb = (k + 1) % 2
            slot = k // BLK
            r = k % BLK
            if k == 5:
                pltpu.make_async_copy(
                    sd_hbm.at[pl.ds(0, BLK)], sdblks[1], isem).wait()
            if k == 8:
                gn = jnp.minimum(2 * m + 2, 2 * NSB - 1)
                pltpu.async_copy(
                    sd_hbm.at[pl.ds(base + gn * BLK, BLK)], sdblks[0], isem)
            if k == 13:
                pltpu.make_async_copy(
                    sd_hbm.at[pl.ds(0, BLK)], sdblks[0], isem).wait()
            pltpu.make_async_copy(
                y_hbm.at[pl.ds(0, C32)], bufs[b], gsems[b]).wait()
            for q in range(2):
                d = sdblks[slot][r, pl.ds(C32 + q * 16, 16)]
                loc = d - coff
                bad = (loc < 0) | (loc >= HALF)
                dls[b][0, pl.ds(q * 16, 16)] = jnp.where(bad, HALF, loc)
            pltpu.async_copy(bufs[b], acc_sh.at[dls[b].at[0]], ssems[b],
                             add=True)
            if k == 0:
                @pl.when(m > 0)
                def _drain_first():
                    pltpu.make_async_copy(
                        y_hbm.at[pl.ds(0, C32)], bufs[nb], ssems[nb]).wait()
            else:
                pltpu.make_async_copy(
                    y_hbm.at[pl.ds(0, C32)], bufs[nb], ssems[nb]).wait()
            if k < 7:
                nref, nrow = sdblks[0], k + 1
            elif k < 15:
                nref, nrow = sdblks[1], k - 7
            else:
                nref, nrow = sdblks[0], 0
            pltpu.async_copy(y_sh.at[nref.at[nrow, pl.ds(0, C32)]], bufs[nb],
                             gsems[nb])
        gn = jnp.minimum(2 * m + 3, 2 * NSB - 1)
        pltpu.async_copy(sd_hbm.at[pl.ds(base + gn * BLK, BLK)], sdblks[1], isem)
        return 0

    lax.fori_loop(0, NSB, sb_body, 0)
    pltpu.make_async_copy(sd_hbm.at[pl.ds(0, BLK)], sdblks[1], isem).wait()
    pltpu.make_async_copy(y_hbm.at[pl.ds(0, C32)], bufs[1], ssems[1]).wait()
    pltpu.make_async_copy(y_hbm.at[pl.ds(0, C32)], bufs[0], gsems[0]).wait()
    plsc.subcore_barrier()
    pltpu.sync_copy(acc_sh.at[pl.ds(s * 312, 312)],
                    out_hbm.at[c, pl.ds(s * 312, 312)])

    @pl.when(s < 9)
    def _copy_tail():
        pltpu.sync_copy(acc_sh.at[pl.ds(4992 + s * 8, 8)],
                        out_hbm.at[c, pl.ds(4992 + s * 8, 8)])


def _mm1_body(x_ref, w_ref, degt_ref, o_ref):
    xw = jnp.dot(x_ref[...], w_ref[...], preferred_element_type=jnp.float32)
    deg = jnp.sum(degt_ref[0:N, :], axis=1, keepdims=True) + 1.0
    o_ref[...] = xw * lax.rsqrt(deg)


_mm1 = pl.pallas_call(
    _mm1_body, out_shape=jax.ShapeDtypeStruct((N, D), jnp.float32))


def _c2_body(part_ref, y1_ref, degt_ref, w_ref, b_ref, o_ref):
    deg = jnp.sum(degt_ref[0:N, :], axis=1, keepdims=True) + 1.0
    dinv = lax.rsqrt(deg)
    agg = jnp.concatenate(
        [part_ref[0, 0:HALF, :], part_ref[1, 0:(N - HALF), :]], axis=0)
    z = (agg + y1_ref[...]) * dinv + b_ref[...]
    o_ref[...] = jnp.dot(z, w_ref[...], preferred_element_type=jnp.float32) * dinv


_c2 = pl.pallas_call(
    _c2_body, out_shape=jax.ShapeDtypeStruct((N, D), jnp.float32))


def _c3_body(part_ref, y2_ref, degt_ref, b_ref, o_ref):
    deg = jnp.sum(degt_ref[0:N, :], axis=1, keepdims=True) + 1.0
    dinv = lax.rsqrt(deg)
    agg = jnp.concatenate(
        [part_ref[0, 0:HALF, :], part_ref[1, 0:(N - HALF), :]], axis=0)
    o_ref[...] = (agg + y2_ref[...]) * dinv + b_ref[...]


_c3 = pl.pallas_call(
    _c3_body, out_shape=jax.ShapeDtypeStruct((N, D), jnp.float32))


def kernel(x, pos_edge_index, neg_edge_index, W1, b1, W2, b2):
    ei = jnp.concatenate([pos_edge_index, neg_edge_index], axis=1).astype(jnp.int32)
    src, dst = ei[0], ei[1]
    pad = EPAD - E
    srcaliflat = jnp.concatenate([src, jnp.zeros((pad,), jnp.int32)])
    dstflat = jnp.concatenate([dst, jnp.full((pad,), N, jnp.int32)])
    dstp = dstflat.reshape(ROWS_PAD, CHUNK)
    sd64 = jnp.concatenate([srcaliflat.reshape(ROWS32, C32),
                            dstflat.reshape(ROWS32, C32)], axis=1)

    degp = _deg_kernel(dstp)
    degt = degp.T
    b1r = b1.reshape(1, D)
    b2r = b2.reshape(1, D)

    y1 = _mm1(x, W1, degt)
    part1 = _spmm_kernel(y1, sd64)
    y2 = _c2(part1, y1, degt, W2, b1r)
    part2 = _spmm_kernel(y2, sd64)
    out = _c3(part2, y2, degt, b2r)
    return out
